# baseline (speedup 1.0000x reference)
import jax
import jax.numpy as jnp
from jax import lax
from jax.experimental import pallas as pl
from jax.experimental.pallas import tpu as pltpu
from jax.experimental.pallas import tpu_sc as plsc

N = 10000
E = 320000
D = 128
DE = 16

NW = 32
EPW = E // NW
_f32 = jnp.float32
_bf16 = jnp.bfloat16
_i32 = jnp.int32


_EB = 8000
_HB = 256


def _edge_lin_body(ea_ref, w0_ref, b0_ref, w1_ref, b1_ref,
                   h_ref, ws_ref, wd_ref,
                   c0_ref, c1_ref, a_ref, b_ref):
    ea = ea_ref[...]
    c0_ref[...] = (jnp.dot(ea, w0_ref[...], preferred_element_type=_f32)
                   + b0_ref[...]).astype(_bf16)
    c1_ref[...] = (jnp.dot(ea, w1_ref[...], preferred_element_type=_f32)
                   + b1_ref[...]).astype(_bf16)
    h = h_ref[...]
    a_ref[...] = jnp.dot(h, ws_ref[...], preferred_element_type=_f32)
    b_ref[...] = jnp.dot(h, wd_ref[...], preferred_element_type=_f32)


def _edge_lin(ea, w0, b0, w1, b1, h, ws, wd):
    return pl.pallas_call(
        _edge_lin_body,
        grid=(E // _EB,),
        in_specs=[
            pl.BlockSpec((_EB, DE), lambda i: (i, 0)),
            pl.BlockSpec((DE, D), lambda i: (0, 0)),
            pl.BlockSpec((1, D), lambda i: (0, 0)),
            pl.BlockSpec((DE, D), lambda i: (0, 0)),
            pl.BlockSpec((1, D), lambda i: (0, 0)),
            pl.BlockSpec((_HB, D), lambda i: (i, 0)),
            pl.BlockSpec((D, D), lambda i: (0, 0)),
            pl.BlockSpec((D, D), lambda i: (0, 0)),
        ],
        out_specs=[
            pl.BlockSpec((_EB, D), lambda i: (i, 0)),
            pl.BlockSpec((_EB, D), lambda i: (i, 0)),
            pl.BlockSpec((_HB, D), lambda i: (i, 0)),
            pl.BlockSpec((_HB, D), lambda i: (i, 0)),
        ],
        out_shape=[jax.ShapeDtypeStruct((E, D), _bf16),
                   jax.ShapeDtypeStruct((E, D), _bf16),
                   jax.ShapeDtypeStruct((N, D), _f32),
                   jax.ShapeDtypeStruct((N, D), _f32)],
    )(ea, w0, b0.reshape(1, D), w1, b1.reshape(1, D), h, ws, wd)


_NB = 1000


def _up_ab_body(h_ref, agg_ref, wuh_ref, wua_ref, bu_ref, ws_ref, wd_ref,
                h1_ref, a1_ref, b1_ref):
    aggs = agg_ref[0] + agg_ref[1]
    h1 = jnp.maximum(
        jnp.dot(h_ref[...], wuh_ref[...], preferred_element_type=_f32)
        + jnp.dot(aggs, wua_ref[...], preferred_element_type=_f32)
        + bu_ref[...], 0.0)
    h1_ref[...] = h1
    a1_ref[...] = jnp.dot(h1, ws_ref[...], preferred_element_type=_f32)
    b1_ref[...] = jnp.dot(h1, wd_ref[...], preferred_element_type=_f32)


def _up_ab(h, agg, wuh, wua, bu, ws, wd):
    return pl.pallas_call(
        _up_ab_body,
        grid=(N // _NB,),
        in_specs=[
            pl.BlockSpec((_NB, D), lambda i: (i, 0)),
            pl.BlockSpec((2, _NB, D), lambda i: (0, i, 0)),
            pl.BlockSpec((D, D), lambda i: (0, 0)),
            pl.BlockSpec((D, D), lambda i: (0, 0)),
            pl.BlockSpec((1, D), lambda i: (0, 0)),
            pl.BlockSpec((D, D), lambda i: (0, 0)),
            pl.BlockSpec((D, D), lambda i: (0, 0)),
        ],
        out_specs=[
            pl.BlockSpec((_NB, D), lambda i: (i, 0)),
            pl.BlockSpec((_NB, D), lambda i: (i, 0)),
            pl.BlockSpec((_NB, D), lambda i: (i, 0)),
        ],
        out_shape=[jax.ShapeDtypeStruct((N, D), _f32)] * 3,
    )(h, agg, wuh, wua, bu.reshape(1, D), ws, wd)


def _up_final_body(h_ref, agg_ref, wuh_ref, wua_ref, bu_ref, out_ref):
    aggs = agg_ref[0] + agg_ref[1]
    out_ref[...] = (
        jnp.dot(h_ref[...], wuh_ref[...], preferred_element_type=_f32)
        + jnp.dot(aggs, wua_ref[...], preferred_element_type=_f32)
        + bu_ref[...])


def _up_final(h, agg, wuh, wua, bu):
    return pl.pallas_call(
        _up_final_body,
        grid=(N // _NB,),
        in_specs=[
            pl.BlockSpec((_NB, D), lambda i: (i, 0)),
            pl.BlockSpec((2, _NB, D), lambda i: (0, i, 0)),
            pl.BlockSpec((D, D), lambda i: (0, 0)),
            pl.BlockSpec((D, D), lambda i: (0, 0)),
            pl.BlockSpec((1, D), lambda i: (0, 0)),
        ],
        out_specs=pl.BlockSpec((_NB, D), lambda i: (i, 0)),
        out_shape=jax.ShapeDtypeStruct((N, D), _f32),
    )(h, agg, wuh, wua, bu.reshape(1, D))


K = 40
NCHUNK = EPW // K
NSLOT = 3


def _sc_mpnn_body(a_hbm, b_hbm, c_hbm, src_hbm, dst_hbm, zeros_hbm, out_hbm,
                  abuf0, bbuf0, cbuf0, sidx0, didx0,
                  abuf1, bbuf1, cbuf1, sidx1, didx1,
                  abuf2, bbuf2, cbuf2, sidx2, didx2,
                  agg_sh,
                  ix0, ga0, sc0, ix1, ga1, sc1, ix2, ga2, sc2):
    core = lax.axis_index("c")
    sub = lax.axis_index("s")
    w = sub * 2 + core

    @pl.when(sub == 0)
    def _():
        pltpu.sync_copy(zeros_hbm, agg_sh)
    plsc.subcore_barrier()

    def idx_pairs(g, slot):
        base = w * EPW + g * K
        return [
            (src_hbm.at[pl.ds(base, K)], slot[3]),
            (dst_hbm.at[pl.ds(base, K)], slot[4]),
        ]

    def issue_idx(g, slot):
        for s, d in idx_pairs(g, slot):
            pltpu.async_copy(s, d, slot[5])

    def drain_idx(g, slot):
        for s, d in idx_pairs(g, slot):
            pltpu.make_async_copy(s, d, slot[5]).wait()

    def ga_pairs(g, slot):
        base = w * EPW + g * K
        return [
            (a_hbm.at[slot[3]], slot[0]),
            (b_hbm.at[slot[4]], slot[1]),
            (c_hbm.at[pl.ds(base, K)], slot[2]),
        ]

    def issue_ga(g, slot):
        for s, d in ga_pairs(g, slot):
            pltpu.async_copy(s, d, slot[6])

    def drain_ga(g, slot):
        for s, d in ga_pairs(g, slot):
            pltpu.make_async_copy(s, d, slot[6]).wait()

    def issue_sc(slot):
        pltpu.async_copy(slot[0], agg_sh.at[slot[4]], slot[7], add=True)

    def drain_sc(slot):
        pltpu.make_async_copy(slot[0], agg_sh.at[slot[4]], slot[7]).wait()

    mask = jnp.full((16,), -65536, _i32)

    def compute(slot):
        abuf, bbuf, cbuf = slot[0], slot[1], slot[2]
        ci = cbuf.bitcast(_i32)

        def rows(rr, c2):
            e0 = 2 * rr
            e1 = 2 * rr + 1
            for j in range(D // 16):
                sl = pl.ds(j * 16, 16)
                cw = ci[rr, sl]
                clo = lax.bitcast_convert_type(lax.shift_left(cw, 16), _f32)
                chi = lax.bitcast_convert_type(cw & mask, _f32)
                abuf[e0, sl] = jnp.maximum(abuf[e0, sl] + bbuf[e0, sl] + clo,
                                           0.0)
                abuf[e1, sl] = jnp.maximum(abuf[e1, sl] + bbuf[e1, sl] + chi,
                                           0.0)
            return c2
        lax.fori_loop(0, K // 2, rows, 0)

    slots = ((abuf0, bbuf0, cbuf0, sidx0, didx0, ix0, ga0, sc0),
             (abuf1, bbuf1, cbuf1, sidx1, didx1, ix1, ga1, sc1),
             (abuf2, bbuf2, cbuf2, sidx2, didx2, ix2, ga2, sc2))

    issue_idx(0, slots[0])
    issue_idx(1, slots[1])
    drain_idx(0, slots[0])
    issue_ga(0, slots[0])

    def chunk_body(g, carry):
        def run(cur, mid, nxt):
            drain_ga(g, cur)

            @pl.when(g >= 1)
            def _():
                drain_sc(nxt)

            @pl.when(g + 2 < NCHUNK)
            def _():
                issue_idx(g + 2, nxt)

            @pl.when(g + 1 < NCHUNK)
            def _():
                drain_idx(g + 1, mid)
                issue_ga(g + 1, mid)

            compute(cur)
            issue_sc(cur)

        for r in range(NSLOT):
            @pl.when(g % NSLOT == r)
            def _(r=r):
                run(slots[r], slots[(r + 1) % NSLOT], slots[(r + 2) % NSLOT])
        return carry

    lax.fori_loop(0, NCHUNK, chunk_body, 0)

    drain_sc(slots[(NCHUNK - 1) % NSLOT])

    plsc.subcore_barrier()

    @pl.when(sub < 15)
    def _():
        off = pl.multiple_of(sub * 632, 8)
        pltpu.sync_copy(agg_sh.at[pl.ds(off, 632)],
                        out_hbm.at[core, pl.ds(off, 632)])

    @pl.when(sub == 15)
    def _():
        pltpu.sync_copy(agg_sh.at[pl.ds(9480, 520)],
                        out_hbm.at[core, pl.ds(9480, 520)])


_mpnn_layer_sc = pl.kernel(
    _sc_mpnn_body,
    out_type=jax.ShapeDtypeStruct((2, N, D), _f32),
    mesh=plsc.VectorSubcoreMesh(core_axis_name="c", subcore_axis_name="s"),
    scratch_types=(
        [pltpu.VMEM((K, D), _f32),
         pltpu.VMEM((K, D), _f32),
         pltpu.VMEM((K, D), _bf16),
         pltpu.VMEM((K,), _i32),
         pltpu.VMEM((K,), _i32)] * NSLOT
        + [pltpu.VMEM_SHARED((N, D), _f32)]
        + [pltpu.SemaphoreType.DMA] * (3 * NSLOT)
    ),
)


def kernel(x, edge_index, edge_attr, Wm0, bm0, Wu0, bu0, Wm1, bm1, Wu1, bu1):
    h0 = jnp.squeeze(x, -1)
    src = edge_index[0]
    dst = edge_index[1]
    zeros = jnp.zeros((N, D), _f32)

    c0, c1, a0, b0 = _edge_lin(edge_attr, Wm0[2 * D:], bm0, Wm1[2 * D:], bm1,
                               h0, Wm0[:D], Wm0[D:2 * D])
    agg0 = _mpnn_layer_sc(a0, b0, c0, src, dst, zeros)
    h1, a1, b1 = _up_ab(h0, agg0, Wu0[:D], Wu0[D:], bu0,
                        Wm1[:D], Wm1[D:2 * D])
    agg1 = _mpnn_layer_sc(a1, b1, c1, src, dst, zeros)
    h2 = _up_final(h1, agg1, Wu1[:D], Wu1[D:], bu1)
    return h2[:, :, None]


# speedup vs baseline: 1.0950x; 1.0950x over previous
import jax
import jax.numpy as jnp
from jax import lax
from jax.experimental import pallas as pl
from jax.experimental.pallas import tpu as pltpu
from jax.experimental.pallas import tpu_sc as plsc

N = 10000
E = 320000
D = 128
DE = 16

NW = 32
EPW = E // NW
_f32 = jnp.float32
_i32 = jnp.int32


def _edge_lin_body(ea_ref, w0_ref, b0_ref, w1_ref, b1_ref, c0_ref, c1_ref):
    ea = ea_ref[...]
    c0_ref[...] = jnp.dot(ea, w0_ref[...], preferred_element_type=_f32) + b0_ref[...]
    c1_ref[...] = jnp.dot(ea, w1_ref[...], preferred_element_type=_f32) + b1_ref[...]


_EB = 8000


def _edge_lin(ea, w0, b0, w1, b1):
    return pl.pallas_call(
        _edge_lin_body,
        grid=(E // _EB,),
        in_specs=[
            pl.BlockSpec((_EB, DE), lambda i: (i, 0)),
            pl.BlockSpec((DE, D), lambda i: (0, 0)),
            pl.BlockSpec((1, D), lambda i: (0, 0)),
            pl.BlockSpec((DE, D), lambda i: (0, 0)),
            pl.BlockSpec((1, D), lambda i: (0, 0)),
        ],
        out_specs=[
            pl.BlockSpec((_EB, D), lambda i: (i, 0)),
            pl.BlockSpec((_EB, D), lambda i: (i, 0)),
        ],
        out_shape=[jax.ShapeDtypeStruct((E, D), _f32)] * 2,
    )(ea, w0, b0.reshape(1, D), w1, b1.reshape(1, D))


_NB = 1000


def _ab_body(h_ref, ws_ref, wd_ref, a_ref, b_ref):
    h = h_ref[...]
    a_ref[...] = jnp.dot(h, ws_ref[...], preferred_element_type=_f32)
    b_ref[...] = jnp.dot(h, wd_ref[...], preferred_element_type=_f32)


def _ab(h, ws, wd):
    return pl.pallas_call(
        _ab_body,
        grid=(N // _NB,),
        in_specs=[
            pl.BlockSpec((_NB, D), lambda i: (i, 0)),
            pl.BlockSpec((D, D), lambda i: (0, 0)),
            pl.BlockSpec((D, D), lambda i: (0, 0)),
        ],
        out_specs=[
            pl.BlockSpec((_NB, D), lambda i: (i, 0)),
            pl.BlockSpec((_NB, D), lambda i: (i, 0)),
        ],
        out_shape=[jax.ShapeDtypeStruct((N, D), _f32)] * 2,
    )(h, ws, wd)


def _up_ab_body(h_ref, agg_ref, wuh_ref, wua_ref, bu_ref, ws_ref, wd_ref,
                h1_ref, a1_ref, b1_ref):
    aggs = agg_ref[0] + agg_ref[1]
    h1 = jnp.maximum(
        jnp.dot(h_ref[...], wuh_ref[...], preferred_element_type=_f32)
        + jnp.dot(aggs, wua_ref[...], preferred_element_type=_f32)
        + bu_ref[...], 0.0)
    h1_ref[...] = h1
    a1_ref[...] = jnp.dot(h1, ws_ref[...], preferred_element_type=_f32)
    b1_ref[...] = jnp.dot(h1, wd_ref[...], preferred_element_type=_f32)


def _up_ab(h, agg, wuh, wua, bu, ws, wd):
    return pl.pallas_call(
        _up_ab_body,
        grid=(N // _NB,),
        in_specs=[
            pl.BlockSpec((_NB, D), lambda i: (i, 0)),
            pl.BlockSpec((2, _NB, D), lambda i: (0, i, 0)),
            pl.BlockSpec((D, D), lambda i: (0, 0)),
            pl.BlockSpec((D, D), lambda i: (0, 0)),
            pl.BlockSpec((1, D), lambda i: (0, 0)),
            pl.BlockSpec((D, D), lambda i: (0, 0)),
            pl.BlockSpec((D, D), lambda i: (0, 0)),
        ],
        out_specs=[
            pl.BlockSpec((_NB, D), lambda i: (i, 0)),
            pl.BlockSpec((_NB, D), lambda i: (i, 0)),
            pl.BlockSpec((_NB, D), lambda i: (i, 0)),
        ],
        out_shape=[jax.ShapeDtypeStruct((N, D), _f32)] * 3,
    )(h, agg, wuh, wua, bu.reshape(1, D), ws, wd)


def _up_final_body(h_ref, agg_ref, wuh_ref, wua_ref, bu_ref, out_ref):
    aggs = agg_ref[0] + agg_ref[1]
    out_ref[...] = (
        jnp.dot(h_ref[...], wuh_ref[...], preferred_element_type=_f32)
        + jnp.dot(aggs, wua_ref[...], preferred_element_type=_f32)
        + bu_ref[...])


def _up_final(h, agg, wuh, wua, bu):
    return pl.pallas_call(
        _up_final_body,
        grid=(N // _NB,),
        in_specs=[
            pl.BlockSpec((_NB, D), lambda i: (i, 0)),
            pl.BlockSpec((2, _NB, D), lambda i: (0, i, 0)),
            pl.BlockSpec((D, D), lambda i: (0, 0)),
            pl.BlockSpec((D, D), lambda i: (0, 0)),
            pl.BlockSpec((1, D), lambda i: (0, 0)),
        ],
        out_specs=pl.BlockSpec((_NB, D), lambda i: (i, 0)),
        out_shape=jax.ShapeDtypeStruct((N, D), _f32),
    )(h, agg, wuh, wua, bu.reshape(1, D))


K = 80
NCHUNK = EPW // K
NSLOT = 4


def _sc_mpnn_body(a_hbm, b_hbm, c_hbm, src_hbm, dst_hbm, zeros_hbm, out_hbm,
                  msg0, sidx0, didx0, msg1, sidx1, didx1,
                  msg2, sidx2, didx2, msg3, sidx3, didx3,
                  agg_sh,
                  ic0, ab0, sc0, ic1, ab1, sc1,
                  ic2, ab2, sc2, ic3, ab3, sc3):
    core = lax.axis_index("c")
    sub = lax.axis_index("s")
    w = sub * 2 + core

    @pl.when(sub == 0)
    def _():
        pltpu.sync_copy(zeros_hbm, agg_sh)
    plsc.subcore_barrier()

    def ic_pairs(g, slot):
        msg, sidx, didx = slot[:3]
        base = w * EPW + g * K
        return [
            (src_hbm.at[pl.ds(base, K)], sidx),
            (dst_hbm.at[pl.ds(base, K)], didx),
            (c_hbm.at[pl.ds(base, K)], msg),
        ]

    def issue_ic(g, slot):
        for s, d in ic_pairs(g, slot):
            pltpu.async_copy(s, d, slot[3])

    def drain_ic(g, slot):
        for s, d in ic_pairs(g, slot):
            pltpu.make_async_copy(s, d, slot[3]).wait()

    def issue_a(slot):
        msg, sidx = slot[0], slot[1]
        pltpu.async_copy(a_hbm.at[sidx], msg, slot[4], add=True)

    def drain_a(slot):
        msg, sidx = slot[0], slot[1]
        pltpu.make_async_copy(a_hbm.at[sidx], msg, slot[4]).wait()

    def issue_b(slot):
        msg, didx = slot[0], slot[2]
        pltpu.async_copy(b_hbm.at[didx], msg, slot[4], add=True)

    def drain_b(slot):
        msg, didx = slot[0], slot[2]
        pltpu.make_async_copy(b_hbm.at[didx], msg, slot[4]).wait()

    def issue_sc(slot):
        msg, didx = slot[0], slot[2]
        pltpu.async_copy(msg, agg_sh.at[didx], slot[5], add=True)

    def drain_sc(slot):
        msg, didx = slot[0], slot[2]
        pltpu.make_async_copy(msg, agg_sh.at[didx], slot[5]).wait()

    def relu(slot):
        msg = slot[0]

        def relu_row(e, c2):
            for j in range(D // 16):
                sl = pl.ds(j * 16, 16)
                msg[e, sl] = jnp.maximum(msg[e, sl], 0.0)
            return c2
        lax.fori_loop(0, K, relu_row, 0)

    slots = ((msg0, sidx0, didx0, ic0, ab0, sc0),
             (msg1, sidx1, didx1, ic1, ab1, sc1),
             (msg2, sidx2, didx2, ic2, ab2, sc2),
             (msg3, sidx3, didx3, ic3, ab3, sc3))

    issue_ic(0, slots[0])
    issue_ic(1, slots[1])
    issue_ic(2, slots[2])
    drain_ic(0, slots[0])
    issue_a(slots[0])
    drain_ic(1, slots[1])
    issue_a(slots[1])
    drain_a(slots[0])
    issue_b(slots[0])

    def chunk_body(g, carry):
        def run(s0, s1, s2, s3):
            drain_b(s0)

            @pl.when(g >= 1)
            def _():
                drain_sc(s3)

            @pl.when(g + 3 < NCHUNK)
            def _():
                issue_ic(g + 3, s3)

            @pl.when(g + 1 < NCHUNK)
            def _():
                drain_a(s1)
                issue_b(s1)

            @pl.when(g + 2 < NCHUNK)
            def _():
                drain_ic(g + 2, s2)
                issue_a(s2)

            relu(s0)
            issue_sc(s0)

        for r in range(NSLOT):
            @pl.when(g % NSLOT == r)
            def _(r=r):
                run(slots[r], slots[(r + 1) % NSLOT],
                    slots[(r + 2) % NSLOT], slots[(r + 3) % NSLOT])
        return carry

    lax.fori_loop(0, NCHUNK, chunk_body, 0)

    drain_sc(slots[(NCHUNK - 1) % NSLOT])

    plsc.subcore_barrier()

    @pl.when(sub < 15)
    def _():
        off = pl.multiple_of(sub * 632, 8)
        pltpu.sync_copy(agg_sh.at[pl.ds(off, 632)],
                        out_hbm.at[core, pl.ds(off, 632)])

    @pl.when(sub == 15)
    def _():
        pltpu.sync_copy(agg_sh.at[pl.ds(9480, 520)],
                        out_hbm.at[core, pl.ds(9480, 520)])


_mpnn_layer_sc = pl.kernel(
    _sc_mpnn_body,
    out_type=jax.ShapeDtypeStruct((2, N, D), _f32),
    mesh=plsc.VectorSubcoreMesh(core_axis_name="c", subcore_axis_name="s"),
    scratch_types=(
        [pltpu.VMEM((K, D), _f32),
         pltpu.VMEM((K,), _i32),
         pltpu.VMEM((K,), _i32)] * NSLOT
        + [pltpu.VMEM_SHARED((N, D), _f32)]
        + [pltpu.SemaphoreType.DMA] * (3 * NSLOT)
    ),
)


def kernel(x, edge_index, edge_attr, Wm0, bm0, Wu0, bu0, Wm1, bm1, Wu1, bu1):
    h0 = jnp.squeeze(x, -1)
    src = edge_index[0]
    dst = edge_index[1]
    zeros = jnp.zeros((N, D), _f32)

    c0, c1 = _edge_lin(edge_attr, Wm0[2 * D:], bm0, Wm1[2 * D:], bm1)
    a0, b0 = _ab(h0, Wm0[:D], Wm0[D:2 * D])
    agg0 = _mpnn_layer_sc(a0, b0, c0, src, dst, zeros)
    h1, a1, b1 = _up_ab(h0, agg0, Wu0[:D], Wu0[D:], bu0,
                        Wm1[:D], Wm1[D:2 * D])
    agg1 = _mpnn_layer_sc(a1, b1, c1, src, dst, zeros)
    h2 = _up_final(h1, agg1, Wu1[:D], Wu1[D:], bu1)
    return h2[:, :, None]


# speedup vs baseline: 1.1022x; 1.0066x over previous
import jax
import jax.numpy as jnp
from jax import lax
from jax.experimental import pallas as pl
from jax.experimental.pallas import tpu as pltpu
from jax.experimental.pallas import tpu_sc as plsc

N = 10000
E = 320000
D = 128
DE = 16

NW = 32
EPW = E // NW
_f32 = jnp.float32
_i32 = jnp.int32


_EB = 8000
_HB = 256


def _edge_lin_body(ea_ref, w0_ref, b0_ref, w1_ref, b1_ref,
                   h_ref, ws_ref, wd_ref,
                   c0_ref, c1_ref, a_ref, b_ref):
    ea = ea_ref[...]
    c0_ref[...] = jnp.dot(ea, w0_ref[...], preferred_element_type=_f32) + b0_ref[...]
    c1_ref[...] = jnp.dot(ea, w1_ref[...], preferred_element_type=_f32) + b1_ref[...]
    h = h_ref[...]
    a_ref[...] = jnp.dot(h, ws_ref[...], preferred_element_type=_f32)
    b_ref[...] = jnp.dot(h, wd_ref[...], preferred_element_type=_f32)


def _edge_lin(ea, w0, b0, w1, b1, h, ws, wd):
    return pl.pallas_call(
        _edge_lin_body,
        grid=(E // _EB,),
        in_specs=[
            pl.BlockSpec((_EB, DE), lambda i: (i, 0)),
            pl.BlockSpec((DE, D), lambda i: (0, 0)),
            pl.BlockSpec((1, D), lambda i: (0, 0)),
            pl.BlockSpec((DE, D), lambda i: (0, 0)),
            pl.BlockSpec((1, D), lambda i: (0, 0)),
            pl.BlockSpec((_HB, D), lambda i: (i, 0)),
            pl.BlockSpec((D, D), lambda i: (0, 0)),
            pl.BlockSpec((D, D), lambda i: (0, 0)),
        ],
        out_specs=[
            pl.BlockSpec((_EB, D), lambda i: (i, 0)),
            pl.BlockSpec((_EB, D), lambda i: (i, 0)),
            pl.BlockSpec((_HB, D), lambda i: (i, 0)),
            pl.BlockSpec((_HB, D), lambda i: (i, 0)),
        ],
        out_shape=[jax.ShapeDtypeStruct((E, D), _f32),
                   jax.ShapeDtypeStruct((E, D), _f32),
                   jax.ShapeDtypeStruct((N, D), _f32),
                   jax.ShapeDtypeStruct((N, D), _f32)],
    )(ea, w0, b0.reshape(1, D), w1, b1.reshape(1, D), h, ws, wd)


_NB = 1000


def _up_ab_body(h_ref, agg_ref, wuh_ref, wua_ref, bu_ref, ws_ref, wd_ref,
                h1_ref, a1_ref, b1_ref):
    aggs = agg_ref[0] + agg_ref[1]
    h1 = jnp.maximum(
        jnp.dot(h_ref[...], wuh_ref[...], preferred_element_type=_f32)
        + jnp.dot(aggs, wua_ref[...], preferred_element_type=_f32)
        + bu_ref[...], 0.0)
    h1_ref[...] = h1
    a1_ref[...] = jnp.dot(h1, ws_ref[...], preferred_element_type=_f32)
    b1_ref[...] = jnp.dot(h1, wd_ref[...], preferred_element_type=_f32)


def _up_ab(h, agg, wuh, wua, bu, ws, wd):
    return pl.pallas_call(
        _up_ab_body,
        grid=(N // _NB,),
        in_specs=[
            pl.BlockSpec((_NB, D), lambda i: (i, 0)),
            pl.BlockSpec((2, _NB, D), lambda i: (0, i, 0)),
            pl.BlockSpec((D, D), lambda i: (0, 0)),
            pl.BlockSpec((D, D), lambda i: (0, 0)),
            pl.BlockSpec((1, D), lambda i: (0, 0)),
            pl.BlockSpec((D, D), lambda i: (0, 0)),
            pl.BlockSpec((D, D), lambda i: (0, 0)),
        ],
        out_specs=[
            pl.BlockSpec((_NB, D), lambda i: (i, 0)),
            pl.BlockSpec((_NB, D), lambda i: (i, 0)),
            pl.BlockSpec((_NB, D), lambda i: (i, 0)),
        ],
        out_shape=[jax.ShapeDtypeStruct((N, D), _f32)] * 3,
    )(h, agg, wuh, wua, bu.reshape(1, D), ws, wd)


def _up_final_body(h_ref, agg_ref, wuh_ref, wua_ref, bu_ref, out_ref):
    aggs = agg_ref[0] + agg_ref[1]
    out_ref[...] = (
        jnp.dot(h_ref[...], wuh_ref[...], preferred_element_type=_f32)
        + jnp.dot(aggs, wua_ref[...], preferred_element_type=_f32)
        + bu_ref[...])


def _up_final(h, agg, wuh, wua, bu):
    return pl.pallas_call(
        _up_final_body,
        grid=(N // _NB,),
        in_specs=[
            pl.BlockSpec((_NB, D), lambda i: (i, 0)),
            pl.BlockSpec((2, _NB, D), lambda i: (0, i, 0)),
            pl.BlockSpec((D, D), lambda i: (0, 0)),
            pl.BlockSpec((D, D), lambda i: (0, 0)),
            pl.BlockSpec((1, D), lambda i: (0, 0)),
        ],
        out_specs=pl.BlockSpec((_NB, D), lambda i: (i, 0)),
        out_shape=jax.ShapeDtypeStruct((N, D), _f32),
    )(h, agg, wuh, wua, bu.reshape(1, D))


K = 80
NCHUNK = EPW // K
NSLOT = 4


def _sc_mpnn_body(a_hbm, b_hbm, c_hbm, src_hbm, dst_hbm, zeros_hbm, out_hbm,
                  msg0, sidx0, didx0, msg1, sidx1, didx1,
                  msg2, sidx2, didx2, msg3, sidx3, didx3,
                  agg_sh,
                  ic0, ab0, sc0, ic1, ab1, sc1,
                  ic2, ab2, sc2, ic3, ab3, sc3):
    core = lax.axis_index("c")
    sub = lax.axis_index("s")
    w = sub * 2 + core

    @pl.when(sub == 0)
    def _():
        pltpu.sync_copy(zeros_hbm, agg_sh)
    plsc.subcore_barrier()

    def ic_pairs(g, slot):
        msg, sidx, didx = slot[:3]
        base = w * EPW + g * K
        return [
            (src_hbm.at[pl.ds(base, K)], sidx),
            (dst_hbm.at[pl.ds(base, K)], didx),
            (c_hbm.at[pl.ds(base, K)], msg),
        ]

    def issue_ic(g, slot):
        for s, d in ic_pairs(g, slot):
            pltpu.async_copy(s, d, slot[3])

    def drain_ic(g, slot):
        for s, d in ic_pairs(g, slot):
            pltpu.make_async_copy(s, d, slot[3]).wait()

    def issue_a(slot):
        msg, sidx = slot[0], slot[1]
        pltpu.async_copy(a_hbm.at[sidx], msg, slot[4], add=True)

    def drain_a(slot):
        msg, sidx = slot[0], slot[1]
        pltpu.make_async_copy(a_hbm.at[sidx], msg, slot[4]).wait()

    def issue_b(slot):
        msg, didx = slot[0], slot[2]
        pltpu.async_copy(b_hbm.at[didx], msg, slot[4], add=True)

    def drain_b(slot):
        msg, didx = slot[0], slot[2]
        pltpu.make_async_copy(b_hbm.at[didx], msg, slot[4]).wait()

    def issue_sc(slot):
        msg, didx = slot[0], slot[2]
        pltpu.async_copy(msg, agg_sh.at[didx], slot[5], add=True)

    def drain_sc(slot):
        msg, didx = slot[0], slot[2]
        pltpu.make_async_copy(msg, agg_sh.at[didx], slot[5]).wait()

    def relu(slot):
        msg = slot[0]

        def relu_row(e, c2):
            for j in range(D // 16):
                sl = pl.ds(j * 16, 16)
                msg[e, sl] = jnp.maximum(msg[e, sl], 0.0)
            return c2
        lax.fori_loop(0, K, relu_row, 0)

    slots = ((msg0, sidx0, didx0, ic0, ab0, sc0),
             (msg1, sidx1, didx1, ic1, ab1, sc1),
             (msg2, sidx2, didx2, ic2, ab2, sc2),
             (msg3, sidx3, didx3, ic3, ab3, sc3))

    issue_ic(0, slots[0])
    issue_ic(1, slots[1])
    issue_ic(2, slots[2])
    drain_ic(0, slots[0])
    issue_a(slots[0])
    drain_ic(1, slots[1])
    issue_a(slots[1])
    drain_a(slots[0])
    issue_b(slots[0])

    def chunk_body(g, carry):
        def run(s0, s1, s2, s3):
            drain_b(s0)

            @pl.when(g >= 1)
            def _():
                drain_sc(s3)

            @pl.when(g + 3 < NCHUNK)
            def _():
                issue_ic(g + 3, s3)

            @pl.when(g + 1 < NCHUNK)
            def _():
                drain_a(s1)
                issue_b(s1)

            @pl.when(g + 2 < NCHUNK)
            def _():
                drain_ic(g + 2, s2)
                issue_a(s2)

            relu(s0)
            issue_sc(s0)

        for r in range(NSLOT):
            @pl.when(g % NSLOT == r)
            def _(r=r):
                run(slots[r], slots[(r + 1) % NSLOT],
                    slots[(r + 2) % NSLOT], slots[(r + 3) % NSLOT])
        return carry

    lax.fori_loop(0, NCHUNK, chunk_body, 0)

    drain_sc(slots[(NCHUNK - 1) % NSLOT])

    plsc.subcore_barrier()

    @pl.when(sub < 15)
    def _():
        off = pl.multiple_of(sub * 632, 8)
        pltpu.sync_copy(agg_sh.at[pl.ds(off, 632)],
                        out_hbm.at[core, pl.ds(off, 632)])

    @pl.when(sub == 15)
    def _():
        pltpu.sync_copy(agg_sh.at[pl.ds(9480, 520)],
                        out_hbm.at[core, pl.ds(9480, 520)])


_mpnn_layer_sc = pl.kernel(
    _sc_mpnn_body,
    out_type=jax.ShapeDtypeStruct((2, N, D), _f32),
    mesh=plsc.VectorSubcoreMesh(core_axis_name="c", subcore_axis_name="s"),
    scratch_types=(
        [pltpu.VMEM((K, D), _f32),
         pltpu.VMEM((K,), _i32),
         pltpu.VMEM((K,), _i32)] * NSLOT
        + [pltpu.VMEM_SHARED((N, D), _f32)]
        + [pltpu.SemaphoreType.DMA] * (3 * NSLOT)
    ),
)


def kernel(x, edge_index, edge_attr, Wm0, bm0, Wu0, bu0, Wm1, bm1, Wu1, bu1):
    h0 = jnp.squeeze(x, -1)
    src = edge_index[0]
    dst = edge_index[1]
    zeros = jnp.zeros((N, D), _f32)

    c0, c1, a0, b0 = _edge_lin(edge_attr, Wm0[2 * D:], bm0, Wm1[2 * D:], bm1,
                               h0, Wm0[:D], Wm0[D:2 * D])
    agg0 = _mpnn_layer_sc(a0, b0, c0, src, dst, zeros)
    h1, a1, b1 = _up_ab(h0, agg0, Wu0[:D], Wu0[D:], bu0,
                        Wm1[:D], Wm1[D:2 * D])
    agg1 = _mpnn_layer_sc(a1, b1, c1, src, dst, zeros)
    h2 = _up_final(h1, agg1, Wu1[:D], Wu1[D:], bu1)
    return h2[:, :, None]


# speedup vs baseline: 1.1029x; 1.0006x over previous
import jax
import jax.numpy as jnp
from jax import lax
from jax.experimental import pallas as pl
from jax.experimental.pallas import tpu as pltpu
from jax.experimental.pallas import tpu_sc as plsc

N = 10000
E = 320000
D = 128
DE = 16

NW = 32
EPW = E // NW
_f32 = jnp.float32
_i32 = jnp.int32


_EB = 8000
_HB = 256


def _edge_lin_body(ea_ref, w0_ref, b0_ref, w1_ref, b1_ref,
                   h_ref, ws_ref, wd_ref,
                   c0_ref, c1_ref, a_ref, b_ref):
    ea = ea_ref[...]
    c0_ref[...] = jnp.dot(ea, w0_ref[...], preferred_element_type=_f32) + b0_ref[...]
    c1_ref[...] = jnp.dot(ea, w1_ref[...], preferred_element_type=_f32) + b1_ref[...]
    h = h_ref[...]
    a_ref[...] = jnp.dot(h, ws_ref[...], preferred_element_type=_f32)
    b_ref[...] = jnp.dot(h, wd_ref[...], preferred_element_type=_f32)


def _edge_lin(ea, w0, b0, w1, b1, h, ws, wd):
    return pl.pallas_call(
        _edge_lin_body,
        grid=(E // _EB,),
        in_specs=[
            pl.BlockSpec((_EB, DE), lambda i: (i, 0)),
            pl.BlockSpec((DE, D), lambda i: (0, 0)),
            pl.BlockSpec((1, D), lambda i: (0, 0)),
            pl.BlockSpec((DE, D), lambda i: (0, 0)),
            pl.BlockSpec((1, D), lambda i: (0, 0)),
            pl.BlockSpec((_HB, D), lambda i: (i, 0)),
            pl.BlockSpec((D, D), lambda i: (0, 0)),
            pl.BlockSpec((D, D), lambda i: (0, 0)),
        ],
        out_specs=[
            pl.BlockSpec((_EB, D), lambda i: (i, 0)),
            pl.BlockSpec((_EB, D), lambda i: (i, 0)),
            pl.BlockSpec((_HB, D), lambda i: (i, 0)),
            pl.BlockSpec((_HB, D), lambda i: (i, 0)),
        ],
        out_shape=[jax.ShapeDtypeStruct((E, D), _f32),
                   jax.ShapeDtypeStruct((E, D), _f32),
                   jax.ShapeDtypeStruct((N, D), _f32),
                   jax.ShapeDtypeStruct((N, D), _f32)],
    )(ea, w0, b0.reshape(1, D), w1, b1.reshape(1, D), h, ws, wd)


_NB = 1000


def _up_ab_body(h_ref, agg_ref, wuh_ref, wua_ref, bu_ref, ws_ref, wd_ref,
                h1_ref, a1_ref, b1_ref):
    aggs = agg_ref[0] + agg_ref[1]
    h1 = jnp.maximum(
        jnp.dot(h_ref[...], wuh_ref[...], preferred_element_type=_f32)
        + jnp.dot(aggs, wua_ref[...], preferred_element_type=_f32)
        + bu_ref[...], 0.0)
    h1_ref[...] = h1
    a1_ref[...] = jnp.dot(h1, ws_ref[...], preferred_element_type=_f32)
    b1_ref[...] = jnp.dot(h1, wd_ref[...], preferred_element_type=_f32)


def _up_ab(h, agg, wuh, wua, bu, ws, wd):
    return pl.pallas_call(
        _up_ab_body,
        grid=(N // _NB,),
        in_specs=[
            pl.BlockSpec((_NB, D), lambda i: (i, 0)),
            pl.BlockSpec((2, _NB, D), lambda i: (0, i, 0)),
            pl.BlockSpec((D, D), lambda i: (0, 0)),
            pl.BlockSpec((D, D), lambda i: (0, 0)),
            pl.BlockSpec((1, D), lambda i: (0, 0)),
            pl.BlockSpec((D, D), lambda i: (0, 0)),
            pl.BlockSpec((D, D), lambda i: (0, 0)),
        ],
        out_specs=[
            pl.BlockSpec((_NB, D), lambda i: (i, 0)),
            pl.BlockSpec((_NB, D), lambda i: (i, 0)),
            pl.BlockSpec((_NB, D), lambda i: (i, 0)),
        ],
        out_shape=[jax.ShapeDtypeStruct((N, D), _f32)] * 3,
    )(h, agg, wuh, wua, bu.reshape(1, D), ws, wd)


def _up_final_body(h_ref, agg_ref, wuh_ref, wua_ref, bu_ref, out_ref):
    aggs = agg_ref[0] + agg_ref[1]
    out_ref[...] = (
        jnp.dot(h_ref[...], wuh_ref[...], preferred_element_type=_f32)
        + jnp.dot(aggs, wua_ref[...], preferred_element_type=_f32)
        + bu_ref[...])


def _up_final(h, agg, wuh, wua, bu):
    return pl.pallas_call(
        _up_final_body,
        grid=(N // _NB,),
        in_specs=[
            pl.BlockSpec((_NB, D), lambda i: (i, 0)),
            pl.BlockSpec((2, _NB, D), lambda i: (0, i, 0)),
            pl.BlockSpec((D, D), lambda i: (0, 0)),
            pl.BlockSpec((D, D), lambda i: (0, 0)),
            pl.BlockSpec((1, D), lambda i: (0, 0)),
        ],
        out_specs=pl.BlockSpec((_NB, D), lambda i: (i, 0)),
        out_shape=jax.ShapeDtypeStruct((N, D), _f32),
    )(h, agg, wuh, wua, bu.reshape(1, D))


K = 80
NCHUNK = EPW // K
NSLOT = 4


def _sc_mpnn_body(a_hbm, b_hbm, c_hbm, src_hbm, dst_hbm, zeros_hbm, out_hbm,
                  msg0, sidx0, didx0, msg1, sidx1, didx1,
                  msg2, sidx2, didx2, msg3, sidx3, didx3,
                  agg_sh,
                  ic0, ab0, sc0, ic1, ab1, sc1,
                  ic2, ab2, sc2, ic3, ab3, sc3):
    core = lax.axis_index("c")
    sub = lax.axis_index("s")
    w = sub * 2 + core

    @pl.when(sub == 0)
    def _():
        pltpu.sync_copy(zeros_hbm, agg_sh)
    plsc.subcore_barrier()

    def ic_pairs(g, slot):
        msg, sidx, didx = slot[:3]
        base = w * EPW + g * K
        return [
            (src_hbm.at[pl.ds(base, K)], sidx),
            (dst_hbm.at[pl.ds(base, K)], didx),
            (c_hbm.at[pl.ds(base, K)], msg),
        ]

    def issue_ic(g, slot):
        for s, d in ic_pairs(g, slot):
            pltpu.async_copy(s, d, slot[3])

    def drain_ic(g, slot):
        for s, d in ic_pairs(g, slot):
            pltpu.make_async_copy(s, d, slot[3]).wait()

    def issue_a(slot):
        msg, sidx = slot[0], slot[1]
        pltpu.async_copy(a_hbm.at[sidx], msg, slot[4], add=True)

    def drain_a(slot):
        msg, sidx = slot[0], slot[1]
        pltpu.make_async_copy(a_hbm.at[sidx], msg, slot[4]).wait()

    def issue_b(slot):
        msg, didx = slot[0], slot[2]
        pltpu.async_copy(b_hbm.at[didx], msg, slot[4], add=True)

    def drain_b(slot):
        msg, didx = slot[0], slot[2]
        pltpu.make_async_copy(b_hbm.at[didx], msg, slot[4]).wait()

    def issue_sc(slot):
        msg, didx = slot[0], slot[2]
        pltpu.async_copy(msg, agg_sh.at[didx], slot[5], add=True)

    def drain_sc(slot):
        msg, didx = slot[0], slot[2]
        pltpu.make_async_copy(msg, agg_sh.at[didx], slot[5]).wait()

    def relu(slot):
        msg = slot[0]

        def relu_row(e, c2):
            for j in range(D // 16):
                sl = pl.ds(j * 16, 16)
                msg[e, sl] = jnp.maximum(msg[e, sl], 0.0)
            return c2
        lax.fori_loop(0, K, relu_row, 0)

    slots = ((msg0, sidx0, didx0, ic0, ab0, sc0),
             (msg1, sidx1, didx1, ic1, ab1, sc1),
             (msg2, sidx2, didx2, ic2, ab2, sc2),
             (msg3, sidx3, didx3, ic3, ab3, sc3))

    issue_ic(0, slots[0])
    issue_ic(1, slots[1])
    issue_ic(2, slots[2])
    drain_ic(0, slots[0])
    issue_a(slots[0])
    drain_ic(1, slots[1])
    issue_a(slots[1])
    drain_a(slots[0])
    issue_b(slots[0])

    def chunk_body(g, carry):
        def run(s0, s1, s2, s3):
            drain_b(s0)

            @pl.when(g >= 1)
            def _():
                drain_sc(s3)

            @pl.when(g + 3 < NCHUNK)
            def _():
                issue_ic(g + 3, s3)

            @pl.when(g + 1 < NCHUNK)
            def _():
                drain_a(s1)
                issue_b(s1)

            @pl.when(g + 2 < NCHUNK)
            def _():
                drain_ic(g + 2, s2)
                issue_a(s2)

            relu(s0)
            issue_sc(s0)

        for r in range(NSLOT):
            @pl.when(g % NSLOT == r)
            def _(r=r):
                run(slots[r], slots[(r + 1) % NSLOT],
                    slots[(r + 2) % NSLOT], slots[(r + 3) % NSLOT])
        return carry

    lax.fori_loop(0, NCHUNK, chunk_body, 0)

    drain_sc(slots[(NCHUNK - 1) % NSLOT])

    plsc.subcore_barrier()

    @pl.when(sub < 15)
    def _():
        off = pl.multiple_of(sub * 632, 8)
        pltpu.sync_copy(agg_sh.at[pl.ds(off, 632)],
                        out_hbm.at[core, pl.ds(off, 632)])

    @pl.when(sub == 15)
    def _():
        pltpu.sync_copy(agg_sh.at[pl.ds(9480, 520)],
                        out_hbm.at[core, pl.ds(9480, 520)])


_MPNN_SC = None


def _mpnn_layer_sc(*args):
    global _MPNN_SC
    if _MPNN_SC is None:
        _MPNN_SC = pl.kernel(
            _sc_mpnn_body,
            out_type=jax.ShapeDtypeStruct((2, N, D), _f32),
            mesh=plsc.VectorSubcoreMesh(core_axis_name="c",
                                        subcore_axis_name="s"),
            scratch_types=(
                [pltpu.VMEM((K, D), _f32),
                 pltpu.VMEM((K,), _i32),
                 pltpu.VMEM((K,), _i32)] * NSLOT
                + [pltpu.VMEM_SHARED((N, D), _f32)]
                + [pltpu.SemaphoreType.DMA] * (3 * NSLOT)
            ),
        )
    return _MPNN_SC(*args)


def kernel(x, edge_index, edge_attr, Wm0, bm0, Wu0, bu0, Wm1, bm1, Wu1, bu1):
    h0 = jnp.squeeze(x, -1)
    src = edge_index[0]
    dst = edge_index[1]
    zeros = jnp.zeros((N, D), _f32)

    c0, c1, a0, b0 = _edge_lin(edge_attr, Wm0[2 * D:], bm0, Wm1[2 * D:], bm1,
                               h0, Wm0[:D], Wm0[D:2 * D])
    agg0 = _mpnn_layer_sc(a0, b0, c0, src, dst, zeros)
    h1, a1, b1 = _up_ab(h0, agg0, Wu0[:D], Wu0[D:], bu0,
                        Wm1[:D], Wm1[D:2 * D])
    agg1 = _mpnn_layer_sc(a1, b1, c1, src, dst, zeros)
    h2 = _up_final(h1, agg1, Wu1[:D], Wu1[D:], bu1)
    return h2[:, :, None]
